# Initial kernel scaffold; baseline (speedup 1.0000x reference)
import functools

import jax
import jax.numpy as jnp
from jax import lax
from jax.experimental import pallas as pl
from jax.experimental.pallas import tpu as pltpu
from jax.experimental.pallas import tpu_sc as plsc

_B, _N, _M, _C, _O, _K = 8, 4096, 1024, 128, 128, 16
_RADIUS = 10.0
_PD = 16
_NC, _NS = 2, 16
_NW = _NC * _NS
_CH = 128


def _sc_mesh():
    return plsc.VectorSubcoreMesh(
        core_axis_name="c", subcore_axis_name="s",
        num_cores=_NC, num_subcores=_NS)


def _gather_rows_sc(table2d, idx_flat, width, rows_per_batch):
    rows = idx_flat.shape[0]
    rows_per_w = rows // _NW
    nch = rows_per_w // _CH

    @functools.partial(
        pl.kernel,
        out_type=jax.ShapeDtypeStruct((rows, width), jnp.float32),
        mesh=_sc_mesh(),
        scratch_types=[
            pltpu.VMEM((_CH,), jnp.int32),
            pltpu.VMEM((_CH, width), jnp.float32),
            pltpu.SemaphoreType.DMA,
        ],
    )
    def k(tab_hbm, idx_hbm, out_hbm, idx_v, rows_v, sem):
        wid = lax.axis_index("s") * _NC + lax.axis_index("c")
        boff = (wid * rows_per_w // rows_per_batch) * _N
        for t in range(nch):
            base = wid * rows_per_w + t * _CH
            pltpu.sync_copy(idx_hbm.at[pl.ds(base, _CH)], idx_v)
            for j in range(_CH // 16):
                sl = pl.ds(j * 16, 16)
                idx_v[sl] = idx_v[sl] + boff
            pltpu.async_copy(tab_hbm.at[idx_v], rows_v, sem).wait()
            pltpu.sync_copy(rows_v, out_hbm.at[pl.ds(base, _CH)])

    return k(table2d, idx_flat)


def _tc_knn(centers_pad, points_t):
    mt = 128

    def body(c_ref, p_ref, o_ref):
        c = c_ref[0]
        pt = p_ref[0]
        a_sq = jnp.sum(c * c, axis=1, keepdims=True)
        b_sq = jnp.sum(pt * pt, axis=0, keepdims=True)
        inner = jnp.dot(c, pt, preferred_element_type=jnp.float32)
        d2 = jnp.maximum(a_sq + b_sq - 2.0 * inner, 0.0)
        dist = jnp.sqrt(d2 + 1e-6)
        vals = jnp.where(dist <= _RADIUS, dist, jnp.float32(1e9))
        iota = lax.broadcasted_iota(jnp.int32, (mt, _N), 1)
        cols = []
        for _ in range(_K):
            m = jnp.min(vals, axis=1, keepdims=True)
            im = jnp.min(jnp.where(vals == m, iota, _N), axis=1, keepdims=True)
            cols.append(im)
            vals = jnp.where(iota == im, jnp.float32("inf"), vals)
        o_ref[0] = jnp.concatenate(cols, axis=1)

    return pl.pallas_call(
        body,
        grid=(_B, _M // mt),
        in_specs=[
            pl.BlockSpec((1, mt, _PD), lambda b, i: (b, i, 0)),
            pl.BlockSpec((1, _PD, _N), lambda b, i: (b, 0, 0)),
        ],
        out_specs=pl.BlockSpec((1, mt, _K), lambda b, i: (b, i, 0)),
        out_shape=jax.ShapeDtypeStruct((_B, _M, _K), jnp.int32),
    )(centers_pad, points_t)


def _tc_mlp(neigh_rows, cf, ln1_g, ln1_b, W1, b1, W2, b2, Wm, bm, res_scale,
            ln2_g, ln2_b):
    mt = 128
    rows = mt * _K

    def body(n_ref, cf_ref, g1_ref, bb1_ref, w1_ref, b1_ref, w2_ref, b2_ref,
             wm_ref, bm_ref, rs_ref, g2_ref, bb2_ref, o_ref):
        nb = n_ref[0]
        cfb = cf_ref[0]
        cfe = jnp.reshape(
            jnp.broadcast_to(cfb[:, None, :], (mt, _K, _C)), (rows, _C))
        combo = jnp.concatenate([cfe, nb], axis=1)
        mu = jnp.mean(combo, axis=1, keepdims=True)
        var = jnp.mean((combo - mu) ** 2, axis=1, keepdims=True)
        cn = (combo - mu) / jnp.sqrt(var + 1e-5) * g1_ref[0] + bb1_ref[0]
        h = jnp.maximum(
            jnp.dot(cn, w1_ref[...], preferred_element_type=jnp.float32)
            + b1_ref[0], 0.0)
        cw = jax.nn.sigmoid(
            jnp.dot(h, w2_ref[...], preferred_element_type=jnp.float32)
            + b2_ref[0])
        w = nb * cw
        wm = jnp.mean(jnp.reshape(w, (mt, _K, _C)), axis=1)
        fused = cfb + wm * rs_ref[0, 0]
        o = jnp.maximum(
            jnp.dot(fused, wm_ref[...], preferred_element_type=jnp.float32)
            + bm_ref[0], 0.0)
        mu2 = jnp.mean(o, axis=1, keepdims=True)
        var2 = jnp.mean((o - mu2) ** 2, axis=1, keepdims=True)
        o_ref[0] = (o - mu2) / jnp.sqrt(var2 + 1e-5) * g2_ref[0] + bb2_ref[0]

    def full(shape):
        return pl.BlockSpec(shape, lambda b, i: tuple(0 for _ in shape))

    return pl.pallas_call(
        body,
        grid=(_B, _M // mt),
        in_specs=[
            pl.BlockSpec((1, rows, _C), lambda b, i: (b, i, 0)),
            pl.BlockSpec((1, mt, _C), lambda b, i: (b, i, 0)),
            full((1, 2 * _C)),
            full((1, 2 * _C)),
            full((2 * _C, _C)),
            full((1, _C)),
            full((_C, _C)),
            full((1, _C)),
            full((_C, _O)),
            full((1, _O)),
            full((1, 1)),
            full((1, _O)),
            full((1, _O)),
        ],
        out_specs=pl.BlockSpec((1, mt, _O), lambda b, i: (b, i, 0)),
        out_shape=jax.ShapeDtypeStruct((_B, _M, _O), jnp.float32),
    )(neigh_rows, cf, ln1_g.reshape(1, -1), ln1_b.reshape(1, -1), W1,
      b1.reshape(1, -1), W2, b2.reshape(1, -1), Wm, bm.reshape(1, -1),
      res_scale.reshape(1, 1), ln2_g.reshape(1, -1), ln2_b.reshape(1, -1))


def kernel(points, feats, center_idx, ln1_g, ln1_b, W1, b1, W2, b2, Wm, bm,
           res_scale, ln2_g, ln2_b):
    points_pad = jnp.pad(points, ((0, 0), (0, 0), (0, _PD - 3)))
    points_pad2d = points_pad.reshape(_B * _N, _PD)
    feats2d = feats.reshape(_B * _N, _C)
    ci_flat = center_idx.reshape(_B * _M)

    centers_rows = _gather_rows_sc(points_pad2d, ci_flat, _PD, _M)
    cf_rows = _gather_rows_sc(feats2d, ci_flat, _C, _M)
    centers_pad = centers_rows.reshape(_B, _M, _PD)
    cf = cf_rows.reshape(_B, _M, _C)

    points_t = jnp.transpose(points_pad, (0, 2, 1))
    knn_idx = _tc_knn(centers_pad, points_t)

    knn_flat = knn_idx.reshape(_B * _M * _K)
    neigh_rows = _gather_rows_sc(feats2d, knn_flat, _C, _M * _K)
    neigh = neigh_rows.reshape(_B, _M * _K, _C)

    out = _tc_mlp(neigh, cf, ln1_g, ln1_b, W1, b1, W2, b2, Wm, bm,
                  res_scale, ln2_g, ln2_b)
    return out, knn_idx


# speedup vs baseline: 10.8198x; 10.8198x over previous
import functools

import jax
import jax.numpy as jnp
from jax import lax
from jax.experimental import pallas as pl
from jax.experimental.pallas import tpu as pltpu
from jax.experimental.pallas import tpu_sc as plsc

_B, _N, _M, _C, _O, _K = 8, 4096, 1024, 128, 128, 16
_RADIUS = 10.0
_PD = 16
_NC, _NS = 2, 16
_NW = _NC * _NS
_CH = 128


def _sc_mesh():
    return plsc.VectorSubcoreMesh(
        core_axis_name="c", subcore_axis_name="s",
        num_cores=_NC, num_subcores=_NS)


def _gather_rows_sc(table2d, idx_flat, width, rows_per_batch):
    rows = idx_flat.shape[0]
    rows_per_w = rows // _NW
    nch = rows_per_w // _CH

    @functools.partial(
        pl.kernel,
        out_type=jax.ShapeDtypeStruct((rows, width), jnp.float32),
        mesh=_sc_mesh(),
        scratch_types=[
            pltpu.VMEM((_CH,), jnp.int32),
            pltpu.VMEM((_CH, width), jnp.float32),
            pltpu.SemaphoreType.DMA,
        ],
    )
    def k(tab_hbm, idx_hbm, out_hbm, idx_v, rows_v, sem):
        wid = lax.axis_index("s") * _NC + lax.axis_index("c")
        boff = (wid * rows_per_w // rows_per_batch) * _N
        for t in range(nch):
            base = wid * rows_per_w + t * _CH
            pltpu.sync_copy(idx_hbm.at[pl.ds(base, _CH)], idx_v)
            for j in range(_CH // 16):
                sl = pl.ds(j * 16, 16)
                idx_v[sl] = idx_v[sl] + boff
            pltpu.async_copy(tab_hbm.at[idx_v], rows_v, sem).wait()
            pltpu.sync_copy(rows_v, out_hbm.at[pl.ds(base, _CH)])

    return k(table2d, idx_flat)


def _tc_knn(centers_pad, points_t):
    mt = 128

    def body(c_ref, p_ref, o_ref):
        c = c_ref[0]
        pt = p_ref[0]
        a_sq = jnp.sum(c * c, axis=1, keepdims=True)
        b_sq = jnp.sum(pt * pt, axis=0, keepdims=True)
        inner = jnp.dot(c, pt, preferred_element_type=jnp.float32)
        d2 = jnp.maximum(a_sq + b_sq - 2.0 * inner, 0.0)
        dist = jnp.sqrt(d2 + 1e-6)
        vals = jnp.where(dist <= _RADIUS, dist, jnp.float32(1e9))
        iota = lax.broadcasted_iota(jnp.int32, (mt, _N), 1)
        cols = []
        for _ in range(_K):
            m = jnp.min(vals, axis=1, keepdims=True)
            im = jnp.min(jnp.where(vals == m, iota, _N), axis=1, keepdims=True)
            cols.append(im)
            vals = jnp.where(iota == im, jnp.float32("inf"), vals)
        o_ref[0] = jnp.concatenate(cols, axis=1)

    return pl.pallas_call(
        body,
        grid=(_B, _M // mt),
        in_specs=[
            pl.BlockSpec((1, mt, _PD), lambda b, i: (b, i, 0)),
            pl.BlockSpec((1, _PD, _N), lambda b, i: (b, 0, 0)),
        ],
        out_specs=pl.BlockSpec((1, mt, _K), lambda b, i: (b, i, 0)),
        out_shape=jax.ShapeDtypeStruct((_B, _M, _K), jnp.int32),
    )(centers_pad, points_t)


def _tc_mlp(neigh_rows, cf, ln1_g, ln1_b, W1, b1, W2, b2, Wm, bm, res_scale,
            ln2_g, ln2_b):
    mt = 128
    rows = mt * _K

    def body(n_ref, cf_ref, g1_ref, bb1_ref, w1_ref, b1_ref, w2_ref, b2_ref,
             wm_ref, bm_ref, rs_ref, g2_ref, bb2_ref, o_ref):
        nb = n_ref[0]
        cfb = cf_ref[0]
        cfe = jnp.reshape(
            jnp.broadcast_to(cfb[:, None, :], (mt, _K, _C)), (rows, _C))
        combo = jnp.concatenate([cfe, nb], axis=1)
        mu = jnp.mean(combo, axis=1, keepdims=True)
        var = jnp.mean((combo - mu) ** 2, axis=1, keepdims=True)
        cn = (combo - mu) / jnp.sqrt(var + 1e-5) * g1_ref[0] + bb1_ref[0]
        h = jnp.maximum(
            jnp.dot(cn, w1_ref[...], preferred_element_type=jnp.float32)
            + b1_ref[0], 0.0)
        cw = jax.nn.sigmoid(
            jnp.dot(h, w2_ref[...], preferred_element_type=jnp.float32)
            + b2_ref[0])
        w = nb * cw
        wm = jnp.mean(jnp.reshape(w, (mt, _K, _C)), axis=1)
        fused = cfb + wm * rs_ref[0, 0]
        o = jnp.maximum(
            jnp.dot(fused, wm_ref[...], preferred_element_type=jnp.float32)
            + bm_ref[0], 0.0)
        mu2 = jnp.mean(o, axis=1, keepdims=True)
        var2 = jnp.mean((o - mu2) ** 2, axis=1, keepdims=True)
        o_ref[0] = (o - mu2) / jnp.sqrt(var2 + 1e-5) * g2_ref[0] + bb2_ref[0]

    def full(shape):
        return pl.BlockSpec(shape, lambda b, i: tuple(0 for _ in shape))

    return pl.pallas_call(
        body,
        grid=(_B, _M // mt),
        in_specs=[
            pl.BlockSpec((1, rows, _C), lambda b, i: (b, i, 0)),
            pl.BlockSpec((1, mt, _C), lambda b, i: (b, i, 0)),
            full((1, 2 * _C)),
            full((1, 2 * _C)),
            full((2 * _C, _C)),
            full((1, _C)),
            full((_C, _C)),
            full((1, _C)),
            full((_C, _O)),
            full((1, _O)),
            full((1, 1)),
            full((1, _O)),
            full((1, _O)),
        ],
        out_specs=pl.BlockSpec((1, mt, _O), lambda b, i: (b, i, 0)),
        out_shape=jax.ShapeDtypeStruct((_B, _M, _O), jnp.float32),
    )(neigh_rows, cf, ln1_g.reshape(1, -1), ln1_b.reshape(1, -1), W1,
      b1.reshape(1, -1), W2, b2.reshape(1, -1), Wm, bm.reshape(1, -1),
      res_scale.reshape(1, 1), ln2_g.reshape(1, -1), ln2_b.reshape(1, -1))


def kernel(points, feats, center_idx, ln1_g, ln1_b, W1, b1, W2, b2, Wm, bm,
           res_scale, ln2_g, ln2_b):
    points_pad = jnp.pad(points, ((0, 0), (0, 0), (0, _PD - 3)))
    points_pad128 = jnp.pad(points, ((0, 0), (0, 0), (0, 128 - 3)))
    points_pad2d = points_pad128.reshape(_B * _N, 128)
    feats2d = feats.reshape(_B * _N, _C)
    ci_flat = center_idx.reshape(_B * _M)

    centers_rows = _gather_rows_sc(points_pad2d, ci_flat, 128, _M)
    cf_rows = _gather_rows_sc(feats2d, ci_flat, _C, _M)
    centers_pad = centers_rows.reshape(_B, _M, 128)[:, :, :_PD]
    cf = cf_rows.reshape(_B, _M, _C)

    points_t = jnp.transpose(points_pad, (0, 2, 1))
    knn_idx = _tc_knn(centers_pad, points_t)

    knn_flat = knn_idx.reshape(_B * _M * _K)
    neigh_rows = _gather_rows_sc(feats2d, knn_flat, _C, _M * _K)
    neigh = neigh_rows.reshape(_B, _M * _K, _C)

    out = _tc_mlp(neigh, cf, ln1_g, ln1_b, W1, b1, W2, b2, Wm, bm,
                  res_scale, ln2_g, ln2_b)
    return out, knn_idx


# speedup vs baseline: 13.2499x; 1.2246x over previous
import functools

import jax
import jax.numpy as jnp
from jax import lax
from jax.experimental import pallas as pl
from jax.experimental.pallas import tpu as pltpu
from jax.experimental.pallas import tpu_sc as plsc

_B, _N, _M, _C, _O, _K = 8, 4096, 1024, 128, 128, 16
_RADIUS = 10.0
_PD = 16
_NC, _NS = 2, 16
_NW = _NC * _NS
_CH = 128


def _sc_mesh():
    return plsc.VectorSubcoreMesh(
        core_axis_name="c", subcore_axis_name="s",
        num_cores=_NC, num_subcores=_NS)


def _gather_rows_sc(table2d, idx_flat, width, rows_per_batch):
    rows = idx_flat.shape[0]
    rows_per_w = rows // _NW
    nch = rows_per_w // _CH

    @functools.partial(
        pl.kernel,
        out_type=jax.ShapeDtypeStruct((rows, width), jnp.float32),
        mesh=_sc_mesh(),
        scratch_types=[
            pltpu.VMEM((_CH,), jnp.int32),
            pltpu.VMEM((_CH, width), jnp.float32),
            pltpu.SemaphoreType.DMA,
        ],
    )
    def k(tab_hbm, idx_hbm, out_hbm, idx_v, rows_v, sem):
        wid = lax.axis_index("s") * _NC + lax.axis_index("c")
        boff = (wid * rows_per_w // rows_per_batch) * _N
        for t in range(nch):
            base = wid * rows_per_w + t * _CH
            pltpu.sync_copy(idx_hbm.at[pl.ds(base, _CH)], idx_v)
            for j in range(_CH // 16):
                sl = pl.ds(j * 16, 16)
                idx_v[sl] = idx_v[sl] + boff
            pltpu.async_copy(tab_hbm.at[idx_v], rows_v, sem).wait()
            pltpu.sync_copy(rows_v, out_hbm.at[pl.ds(base, _CH)])

    return k(table2d, idx_flat)


def _tc_knn(centers_pad, points_t):
    mt = 128

    def body(c_ref, p_ref, o_ref):
        c = c_ref[0]
        pt = p_ref[0]
        a_sq = jnp.sum(c * c, axis=1, keepdims=True)
        b_sq = jnp.sum(pt * pt, axis=0, keepdims=True)
        inner = jnp.dot(c, pt, preferred_element_type=jnp.float32)
        d2 = jnp.maximum(a_sq + b_sq - 2.0 * inner, 0.0)
        dist = jnp.sqrt(d2 + 1e-6)
        vals = jnp.where(dist <= _RADIUS, dist, jnp.float32(1e9))
        fiota = lax.broadcasted_iota(jnp.int32, (mt, _N), 1).astype(jnp.float32)
        cols = []
        for _ in range(_K):
            m = jnp.min(vals, axis=1, keepdims=True)
            im = jnp.min(jnp.where(vals == m, fiota, jnp.float32(_N)),
                         axis=1, keepdims=True)
            cols.append(im)
            vals = jnp.where(fiota == im, jnp.float32("inf"), vals)
        o_ref[0] = jnp.concatenate(cols, axis=1).astype(jnp.int32)

    return pl.pallas_call(
        body,
        grid=(_B, _M // mt),
        in_specs=[
            pl.BlockSpec((1, mt, _PD), lambda b, i: (b, i, 0)),
            pl.BlockSpec((1, _PD, _N), lambda b, i: (b, 0, 0)),
        ],
        out_specs=pl.BlockSpec((1, mt, _K), lambda b, i: (b, i, 0)),
        out_shape=jax.ShapeDtypeStruct((_B, _M, _K), jnp.int32),
    )(centers_pad, points_t)


def _tc_mlp(neigh_rows, cf, ln1_g, ln1_b, W1, b1, W2, b2, Wm, bm, res_scale,
            ln2_g, ln2_b):
    mt = 128
    rows = mt * _K

    def body(n_ref, cf_ref, g1_ref, bb1_ref, w1_ref, b1_ref, w2_ref, b2_ref,
             wm_ref, bm_ref, rs_ref, g2_ref, bb2_ref, o_ref):
        nb = n_ref[0]
        cfb = cf_ref[0]
        cfe = jnp.reshape(
            jnp.broadcast_to(cfb[:, None, :], (mt, _K, _C)), (rows, _C))
        combo = jnp.concatenate([cfe, nb], axis=1)
        mu = jnp.mean(combo, axis=1, keepdims=True)
        var = jnp.mean((combo - mu) ** 2, axis=1, keepdims=True)
        cn = (combo - mu) / jnp.sqrt(var + 1e-5) * g1_ref[0] + bb1_ref[0]
        h = jnp.maximum(
            jnp.dot(cn, w1_ref[...], preferred_element_type=jnp.float32)
            + b1_ref[0], 0.0)
        cw = jax.nn.sigmoid(
            jnp.dot(h, w2_ref[...], preferred_element_type=jnp.float32)
            + b2_ref[0])
        w = nb * cw
        wm = jnp.mean(jnp.reshape(w, (mt, _K, _C)), axis=1)
        fused = cfb + wm * rs_ref[0, 0]
        o = jnp.maximum(
            jnp.dot(fused, wm_ref[...], preferred_element_type=jnp.float32)
            + bm_ref[0], 0.0)
        mu2 = jnp.mean(o, axis=1, keepdims=True)
        var2 = jnp.mean((o - mu2) ** 2, axis=1, keepdims=True)
        o_ref[0] = (o - mu2) / jnp.sqrt(var2 + 1e-5) * g2_ref[0] + bb2_ref[0]

    def full(shape):
        return pl.BlockSpec(shape, lambda b, i: tuple(0 for _ in shape))

    return pl.pallas_call(
        body,
        grid=(_B, _M // mt),
        in_specs=[
            pl.BlockSpec((1, rows, _C), lambda b, i: (b, i, 0)),
            pl.BlockSpec((1, mt, _C), lambda b, i: (b, i, 0)),
            full((1, 2 * _C)),
            full((1, 2 * _C)),
            full((2 * _C, _C)),
            full((1, _C)),
            full((_C, _C)),
            full((1, _C)),
            full((_C, _O)),
            full((1, _O)),
            full((1, 1)),
            full((1, _O)),
            full((1, _O)),
        ],
        out_specs=pl.BlockSpec((1, mt, _O), lambda b, i: (b, i, 0)),
        out_shape=jax.ShapeDtypeStruct((_B, _M, _O), jnp.float32),
    )(neigh_rows, cf, ln1_g.reshape(1, -1), ln1_b.reshape(1, -1), W1,
      b1.reshape(1, -1), W2, b2.reshape(1, -1), Wm, bm.reshape(1, -1),
      res_scale.reshape(1, 1), ln2_g.reshape(1, -1), ln2_b.reshape(1, -1))


def kernel(points, feats, center_idx, ln1_g, ln1_b, W1, b1, W2, b2, Wm, bm,
           res_scale, ln2_g, ln2_b):
    points_pad = jnp.pad(points, ((0, 0), (0, 0), (0, _PD - 3)))
    points_pad128 = jnp.pad(points, ((0, 0), (0, 0), (0, 128 - 3)))
    points_pad2d = points_pad128.reshape(_B * _N, 128)
    feats2d = feats.reshape(_B * _N, _C)
    ci_flat = center_idx.reshape(_B * _M)

    centers_rows = _gather_rows_sc(points_pad2d, ci_flat, 128, _M)
    cf_rows = _gather_rows_sc(feats2d, ci_flat, _C, _M)
    centers_pad = centers_rows.reshape(_B, _M, 128)[:, :, :_PD]
    cf = cf_rows.reshape(_B, _M, _C)

    points_t = jnp.transpose(points_pad, (0, 2, 1))
    knn_idx = _tc_knn(centers_pad, points_t)

    knn_flat = knn_idx.reshape(_B * _M * _K)
    neigh_rows = _gather_rows_sc(feats2d, knn_flat, _C, _M * _K)
    neigh = neigh_rows.reshape(_B, _M * _K, _C)

    out = _tc_mlp(neigh, cf, ln1_g, ln1_b, W1, b1, W2, b2, Wm, bm,
                  res_scale, ln2_g, ln2_b)
    return out, knn_idx
